# baseline (speedup 1.0000x reference)
import functools
import math

import jax
import jax.numpy as jnp
import numpy as np
from jax.experimental import pallas as pl
from jax.experimental.pallas import tpu as pltpu

_B, _Q, _E = 2, 128, 128
_M, _C, _D = 4096, 32, 128
_S, _K, _H = 128, 8, 4
_HS = _S // _H
_G = 32
_NH = 4
_GH8 = _G // _NH
_R = _GH8 * _K * _C
_GK = _GH8 * _K
_GH = _GH8 * _H
_GKS = _G * _K
_NSTEP = (_B * _Q) // _G


def _pos_enc_np():
    freqs = np.arange(0, _D, 2.0)
    inv_freq = 10000.0 ** (-freqs / _D)
    pos_seq = np.arange(_C - 1, -1, -1.0)
    sinusoid_inp = np.einsum("i,j->ij", pos_seq, inv_freq)
    return np.concatenate(
        [np.sin(sinusoid_inp), np.cos(sinusoid_inp)], axis=-1
    ).astype(np.float32)


def _select_body(q_ref, mk_ref, wq_ref, wk_ref, wqa_ref, bqa_ref, wka_ref,
                 bka_ref, idx_ref, w_ref, u_ref, beta_ref):
    b = pl.program_id(0)
    q = q_ref[0]
    qh = jnp.dot(q, wq_ref[...], preferred_element_type=jnp.float32)
    kh = jnp.dot(mk_ref[0], wk_ref[...], preferred_element_type=jnp.float32)
    logits = jax.lax.dot_general(
        qh, kh, (((1,), (1,)), ((), ())),
        preferred_element_type=jnp.float32) * (1.0 / math.sqrt(_S))
    coln = -jax.lax.broadcasted_iota(jnp.int32, (_Q, _M), 1).astype(
        jnp.float32)
    x = logits
    vals, idxs = [], []
    neg = jnp.float32(-3.0e38)
    for _ in range(_K):
        mx = jnp.max(x, axis=1, keepdims=True)
        ixn = jnp.max(jnp.where(x >= mx, coln, neg), axis=1,
                      keepdims=True)
        vals.append(mx)
        idxs.append(ixn)
        x = jnp.where(coln == ixn, neg, x)
    v = jnp.concatenate(vals, axis=1)
    ix = (-jnp.concatenate(idxs, axis=1)).astype(jnp.int32)
    e = jnp.exp(v - v[:, :1])
    w = e / jnp.sum(e, axis=1, keepdims=True)
    idx_ref[0] = ix + b * _M
    w_ref[0] = w

    qa = (jnp.dot(q, wqa_ref[...], preferred_element_type=jnp.float32)
          + bqa_ref[...])
    lane = jax.lax.broadcasted_iota(jnp.int32, (_Q, _S), 1)
    for h in range(_H):
        qam = jnp.where((lane // _HS) == h, qa, 0.0)
        u_ref[h] = jax.lax.dot_general(
            qam, wka_ref[...], (((1,), (1,)), ((), ())),
            preferred_element_type=jnp.float32)
    sel = ((jax.lax.broadcasted_iota(jnp.int32, (_S, _H), 0) // _HS)
           == jax.lax.broadcasted_iota(jnp.int32, (_S, _H), 1)
           ).astype(jnp.float32)
    beta_ref[...] = jnp.dot(qa * bka_ref[...], sel,
                            preferred_element_type=jnp.float32)


def _select_call(queries, memory_keys, Wq, Wk, Wq_a, bq_a2, Wk_a, bk_a2):
    return pl.pallas_call(
        _select_body,
        grid=(_B,),
        in_specs=[
            pl.BlockSpec((1, _Q, _E), lambda b: (b, 0, 0)),
            pl.BlockSpec((1, _M, _D), lambda b: (b, 0, 0)),
            pl.BlockSpec((_E, _S), lambda b: (0, 0)),
            pl.BlockSpec((_D, _S), lambda b: (0, 0)),
            pl.BlockSpec((_E, _S), lambda b: (0, 0)),
            pl.BlockSpec((1, _S), lambda b: (0, 0)),
            pl.BlockSpec((_D, _S), lambda b: (0, 0)),
            pl.BlockSpec((1, _S), lambda b: (0, 0)),
        ],
        out_specs=[
            pl.BlockSpec((1, _Q, _K), lambda b: (b, 0, 0)),
            pl.BlockSpec((1, _Q, _K), lambda b: (b, 0, 0)),
            pl.BlockSpec((_H, _Q, _D), lambda b: (0, b, 0)),
            pl.BlockSpec((_Q, _H), lambda b: (b, 0)),
        ],
        out_shape=[
            jax.ShapeDtypeStruct((_B, _Q, _K), jnp.int32),
            jax.ShapeDtypeStruct((_B, _Q, _K), jnp.float32),
            jax.ShapeDtypeStruct((_H, _B * _Q, _D), jnp.float32),
            jax.ShapeDtypeStruct((_B * _Q, _H), jnp.float32),
        ],
    )(queries, memory_keys, Wq, Wk, Wq_a, bq_a2, Wk_a, bk_a2)


def _half_attend(chunks, pos, ctile, u2, bqh, w3, seg64, seg64t, vld,
                 wva, bva, boa, woa):
    pu = jax.lax.dot_general(
        u2, pos, (((1,), (1,)), ((), ())),
        preferred_element_type=jnp.float32)
    lraw = jax.lax.dot_general(
        u2, chunks, (((1,), (1,)), ((), ())),
        preferred_element_type=jnp.float32) + jnp.dot(
            pu, ctile, preferred_element_type=jnp.float32)

    asel = ((jax.lax.broadcasted_iota(jnp.int32, (_GH, _GH8), 0) % _GH8)
            == jax.lax.broadcasted_iota(jnp.int32, (_GH, _GH8), 1)
            ).astype(jnp.float32)
    hmask = ((jax.lax.broadcasted_iota(jnp.int32, (_GH, _H), 0) // _GH8)
             == jax.lax.broadcasted_iota(jnp.int32, (_GH, _H), 1)
             ).astype(jnp.float32)
    bcol = jnp.dot(jnp.dot(asel, bqh,
                           preferred_element_type=jnp.float32) * hmask,
                   jnp.ones((_H, 1), jnp.float32),
                   preferred_element_type=jnp.float32)

    lsc = (lraw + bcol) * (1.0 / math.sqrt(_HS))
    e = jnp.exp(lsc) * vld

    w3e = jnp.concatenate([w3] * _GH8, axis=1)
    pmask = ((jax.lax.broadcasted_iota(jnp.int32, (_GH8, _GK), 1) // _K)
             == jax.lax.broadcasted_iota(jnp.int32, (_GH8, _GK), 0)
             ).astype(jnp.float32)
    wrow = jnp.dot(jnp.ones((1, _GH8), jnp.float32), w3e * pmask,
                   preferred_element_type=jnp.float32)

    s = jax.lax.dot_general(
        e, seg64t, (((1,), (0,)), ((), ())),
        preferred_element_type=jnp.float32)
    s2 = s / jnp.maximum(wrow, jnp.float32(1e-30))
    d2 = jnp.dot(s2, seg64,
                 preferred_element_type=jnp.float32)
    cw = e / jnp.maximum(d2, jnp.float32(1e-30))

    csum = jax.lax.dot_general(
        cw, ctile, (((1,), (1,)), ((), ())),
        preferred_element_type=jnp.float32)
    t3 = jnp.dot(cw, chunks,
                 preferred_element_type=jnp.float32) + jnp.dot(
        csum, pos, preferred_element_type=jnp.float32)
    zs = []
    for h in range(_H):
        zs.append(jnp.dot(
            t3[h * _GH8:(h + 1) * _GH8, :], wva[:, h * _HS:(h + 1) * _HS],
            preferred_element_type=jnp.float32))
    z = jnp.concatenate(zs, axis=1) + bva
    return jnp.dot(z, woa,
                   preferred_element_type=jnp.float32) + boa


def _attend_body(idx_ref, mc_ref, pos_ref, u_ref, beta_ref, w3_ref,
                 seg64_ref, seg64t_ref, vld_ref, wva_ref, bva_ref, woa_ref,
                 boa_ref, out_ref, abuf0, abuf1, sem):
    i = pl.program_id(0)

    def _issue(step, buf, sid):
        for j in range(_GKS):
            ix = idx_ref[step * _GKS + j]
            pltpu.make_async_copy(
                mc_ref.at[ix], buf.at[j], sem.at[sid, j // _GK]).start()

    def _compute(buf, sid):
        pos = pos_ref[...]
        ctile = ((jax.lax.broadcasted_iota(jnp.int32, (_C, _R), 1) % _C)
                 == jax.lax.broadcasted_iota(jnp.int32, (_C, _R), 0)
                 ).astype(jnp.float32)
        for nh in range(_NH):
            pltpu.make_async_copy(
                mc_ref.at[pl.ds(0, _GK)],
                buf.at[pl.ds(nh * _GK, _GK)], sem.at[sid, nh]).wait()
            chunks = buf[nh * _GK:(nh + 1) * _GK].reshape(_R, _D)
            u2 = u_ref[:, nh * _GH8:(nh + 1) * _GH8, :].reshape(_GH, _D)
            bqh = beta_ref[nh * _GH8:(nh + 1) * _GH8, :]
            w3 = w3_ref[nh * _GH8:(nh + 1) * _GH8, :]
            out_ref[nh * _GH8:(nh + 1) * _GH8, :] = _half_attend(
                chunks, pos, ctile, u2, bqh, w3, seg64_ref[...],
                seg64t_ref[...], vld_ref[...], wva_ref[...], bva_ref[...],
                boa_ref[...], woa_ref[...])

    @pl.when(i == 0)
    def _():
        _issue(i, abuf0, 0)

    @pl.when(jax.lax.rem(i, 2) == 0)
    def _():
        @pl.when(i + 1 < _NSTEP)
        def _():
            _issue(i + 1, abuf1, 1)

        _compute(abuf0, 0)

    @pl.when(jax.lax.rem(i, 2) == 1)
    def _():
        @pl.when(i + 1 < _NSTEP)
        def _():
            _issue(i + 1, abuf0, 0)

        _compute(abuf1, 1)


def _attend_call(flat_idx, mc_flat, pos, u, beta, w2, seg64, seg64t, vld,
                 Wv_a, bv2, Wo_a, bo2):
    grid_spec = pltpu.PrefetchScalarGridSpec(
        num_scalar_prefetch=1,
        grid=(_NSTEP,),
        in_specs=[
            pl.BlockSpec(memory_space=pl.ANY),
            pl.BlockSpec((_C, _D), lambda g, idx_ref: (0, 0)),
            pl.BlockSpec((_H, _G, _D), lambda g, idx_ref: (0, g, 0)),
            pl.BlockSpec((_G, _H), lambda g, idx_ref: (g, 0)),
            pl.BlockSpec((_G, _K), lambda g, idx_ref: (g, 0)),
            pl.BlockSpec((_GK, _R), lambda g, idx_ref: (0, 0)),
            pl.BlockSpec((_R, _GK), lambda g, idx_ref: (0, 0)),
            pl.BlockSpec((_GH, _R), lambda g, idx_ref: (0, 0)),
            pl.BlockSpec((_D, _S), lambda g, idx_ref: (0, 0)),
            pl.BlockSpec((1, _S), lambda g, idx_ref: (0, 0)),
            pl.BlockSpec((_S, _S), lambda g, idx_ref: (0, 0)),
            pl.BlockSpec((1, _S), lambda g, idx_ref: (0, 0)),
        ],
        out_specs=pl.BlockSpec((_G, _S), lambda g, idx_ref: (g, 0)),
        scratch_shapes=[
            pltpu.VMEM((_GKS, _C, _D), jnp.float32),
            pltpu.VMEM((_GKS, _C, _D), jnp.float32),
            pltpu.SemaphoreType.DMA((2, _NH)),
        ],
    )
    return pl.pallas_call(
        _attend_body,
        grid_spec=grid_spec,
        out_shape=jax.ShapeDtypeStruct((_B * _Q, _S), jnp.float32),
    )(flat_idx, mc_flat, pos, u, beta, w2, seg64, seg64t, vld,
      Wv_a, bv2, Wo_a, bo2)


def _np_consts():
    r = np.arange(_R)
    seg64 = (r[None, :] // _C == np.arange(_GK)[:, None]).astype(np.float32)
    seg64t = seg64.T.copy()
    j = np.arange(_GH)
    vld = ((j[:, None] % _GH8)
           == (r[None, :] // (_K * _C))).astype(np.float32)
    return seg64, seg64t, vld


def kernel(queries, memory_keys, memory_contents, steps_since_last_write,
           accumulator, Wq, Wk, Wq_a, bq_a, Wk_a, bk_a, Wv_a, bv_a, Wo_a,
           bo_a):
    del steps_since_last_write, accumulator
    idx, w, u, beta = _select_call(
        queries, memory_keys, Wq, Wk, Wq_a, bq_a.reshape(1, _S), Wk_a,
        bk_a.reshape(1, _S))
    mc_flat = memory_contents.reshape(_B * _M, _C, _D)
    flat_idx = idx.reshape(-1)
    pos = jnp.asarray(_pos_enc_np())
    seg64, seg64t, vld = _np_consts()
    out = _attend_call(flat_idx, mc_flat, pos, u, beta,
                       w.reshape(_B * _Q, _K), jnp.asarray(seg64),
                       jnp.asarray(seg64t), jnp.asarray(vld), Wv_a,
                       bv_a.reshape(1, _S), Wo_a, bo_a.reshape(1, _S))
    return out.reshape(_B, _Q, _S)


# speedup vs baseline: 1.0690x; 1.0690x over previous
import functools
import math

import jax
import jax.numpy as jnp
import numpy as np
from jax.experimental import pallas as pl
from jax.experimental.pallas import tpu as pltpu

_B, _Q, _E = 2, 128, 128
_M, _C, _D = 4096, 32, 128
_S, _K, _H = 128, 8, 4
_HS = _S // _H
_G = 32
_NH = 4
_GH8 = _G // _NH
_R = _GH8 * _K * _C
_GK = _GH8 * _K
_GH = _GH8 * _H
_GKS = _G * _K
_NSTEP = (_B * _Q) // _G


def _pos_enc_np():
    freqs = np.arange(0, _D, 2.0)
    inv_freq = 10000.0 ** (-freqs / _D)
    pos_seq = np.arange(_C - 1, -1, -1.0)
    sinusoid_inp = np.einsum("i,j->ij", pos_seq, inv_freq)
    return np.concatenate(
        [np.sin(sinusoid_inp), np.cos(sinusoid_inp)], axis=-1
    ).astype(np.float32)


def _select_body(q_ref, mk_ref, wq_ref, wk_ref, wqa_ref, bqa_ref, wka_ref,
                 bka_ref, idx_ref, w_ref, u_ref, beta_ref):
    b = pl.program_id(0)
    q = q_ref[0]
    qh = jnp.dot(q, wq_ref[...], preferred_element_type=jnp.float32)
    kh = jnp.dot(mk_ref[0], wk_ref[...], preferred_element_type=jnp.float32)
    logits = jax.lax.dot_general(
        qh, kh, (((1,), (1,)), ((), ())),
        preferred_element_type=jnp.float32) * (1.0 / math.sqrt(_S))
    coln = -jax.lax.broadcasted_iota(jnp.int32, (_Q, _M), 1).astype(
        jnp.float32)
    x = logits
    vals, idxs = [], []
    neg = jnp.float32(-3.0e38)
    for _ in range(_K):
        mx = jnp.max(x, axis=1, keepdims=True)
        ixn = jnp.max(jnp.where(x >= mx, coln, neg), axis=1,
                      keepdims=True)
        vals.append(mx)
        idxs.append(ixn)
        x = jnp.where(coln == ixn, neg, x)
    v = jnp.concatenate(vals, axis=1)
    ix = (-jnp.concatenate(idxs, axis=1)).astype(jnp.int32)
    e = jnp.exp(v - v[:, :1])
    w = e / jnp.sum(e, axis=1, keepdims=True)
    idx_ref[0] = ix + b * _M
    w_ref[0] = w

    qa = (jnp.dot(q, wqa_ref[...], preferred_element_type=jnp.float32)
          + bqa_ref[...])
    lane = jax.lax.broadcasted_iota(jnp.int32, (_Q, _S), 1)
    for h in range(_H):
        qam = jnp.where((lane // _HS) == h, qa, 0.0)
        u_ref[h] = jax.lax.dot_general(
            qam, wka_ref[...], (((1,), (1,)), ((), ())),
            preferred_element_type=jnp.float32)
    sel = ((jax.lax.broadcasted_iota(jnp.int32, (_S, _H), 0) // _HS)
           == jax.lax.broadcasted_iota(jnp.int32, (_S, _H), 1)
           ).astype(jnp.float32)
    beta_ref[...] = jnp.dot(qa * bka_ref[...], sel,
                            preferred_element_type=jnp.float32)


def _select_call(queries, memory_keys, Wq, Wk, Wq_a, bq_a2, Wk_a, bk_a2):
    return pl.pallas_call(
        _select_body,
        grid=(_B,),
        in_specs=[
            pl.BlockSpec((1, _Q, _E), lambda b: (b, 0, 0)),
            pl.BlockSpec((1, _M, _D), lambda b: (b, 0, 0)),
            pl.BlockSpec((_E, _S), lambda b: (0, 0)),
            pl.BlockSpec((_D, _S), lambda b: (0, 0)),
            pl.BlockSpec((_E, _S), lambda b: (0, 0)),
            pl.BlockSpec((1, _S), lambda b: (0, 0)),
            pl.BlockSpec((_D, _S), lambda b: (0, 0)),
            pl.BlockSpec((1, _S), lambda b: (0, 0)),
        ],
        out_specs=[
            pl.BlockSpec((1, _Q, _K), lambda b: (b, 0, 0)),
            pl.BlockSpec((1, _Q, _K), lambda b: (b, 0, 0)),
            pl.BlockSpec((_H, _Q, _D), lambda b: (0, b, 0)),
            pl.BlockSpec((_Q, _H), lambda b: (b, 0)),
        ],
        out_shape=[
            jax.ShapeDtypeStruct((_B, _Q, _K), jnp.int32),
            jax.ShapeDtypeStruct((_B, _Q, _K), jnp.float32),
            jax.ShapeDtypeStruct((_H, _B * _Q, _D), jnp.float32),
            jax.ShapeDtypeStruct((_B * _Q, _H), jnp.float32),
        ],
    )(queries, memory_keys, Wq, Wk, Wq_a, bq_a2, Wk_a, bk_a2)


def _half_attend(chunks, pos, ctile, u2, bqh, w3, seg64, seg64t, vld,
                 wva, bva, boa, woa):
    pu = jax.lax.dot_general(
        u2, pos, (((1,), (1,)), ((), ())),
        preferred_element_type=jnp.float32)
    lraw = jax.lax.dot_general(
        u2, chunks, (((1,), (1,)), ((), ())),
        preferred_element_type=jnp.float32) + jnp.dot(
            pu, ctile, preferred_element_type=jnp.float32)

    asel = ((jax.lax.broadcasted_iota(jnp.int32, (_GH, _GH8), 0) % _GH8)
            == jax.lax.broadcasted_iota(jnp.int32, (_GH, _GH8), 1)
            ).astype(jnp.float32)
    hmask = ((jax.lax.broadcasted_iota(jnp.int32, (_GH, _H), 0) // _GH8)
             == jax.lax.broadcasted_iota(jnp.int32, (_GH, _H), 1)
             ).astype(jnp.float32)
    bcol = jnp.dot(jnp.dot(asel, bqh,
                           preferred_element_type=jnp.float32) * hmask,
                   jnp.ones((_H, 1), jnp.float32),
                   preferred_element_type=jnp.float32)

    lsc = (lraw + bcol) * (1.0 / math.sqrt(_HS))
    e = jnp.exp(lsc) * vld

    w3e = jnp.concatenate([w3] * _GH8, axis=1)
    pmask = ((jax.lax.broadcasted_iota(jnp.int32, (_GH8, _GK), 1) // _K)
             == jax.lax.broadcasted_iota(jnp.int32, (_GH8, _GK), 0)
             ).astype(jnp.float32)
    wrow = jnp.dot(jnp.ones((1, _GH8), jnp.float32), w3e * pmask,
                   preferred_element_type=jnp.float32)

    s = jax.lax.dot_general(
        e, seg64t, (((1,), (0,)), ((), ())),
        preferred_element_type=jnp.float32)
    s2 = s / jnp.maximum(wrow, jnp.float32(1e-30))
    d2 = jnp.dot(s2, seg64,
                 preferred_element_type=jnp.float32)
    cw = e / jnp.maximum(d2, jnp.float32(1e-30))

    csum = jax.lax.dot_general(
        cw, ctile, (((1,), (1,)), ((), ())),
        preferred_element_type=jnp.float32)
    t3 = jnp.dot(cw, chunks,
                 preferred_element_type=jnp.float32) + jnp.dot(
        csum, pos, preferred_element_type=jnp.float32)
    zs = []
    for h in range(_H):
        zs.append(jnp.dot(
            t3[h * _GH8:(h + 1) * _GH8, :], wva[:, h * _HS:(h + 1) * _HS],
            preferred_element_type=jnp.float32))
    z = jnp.concatenate(zs, axis=1) + bva
    return jnp.dot(z, woa,
                   preferred_element_type=jnp.float32) + boa


def _attend_body(idx_ref, mc_ref, pos_ref, u_ref, beta_ref, w3_ref,
                 seg64_ref, seg64t_ref, vld_ref, wva_ref, bva_ref, woa_ref,
                 boa_ref, out_ref, abuf0, abuf1, sem):
    i = pl.program_id(0)

    def _issue(step, buf, sid):
        for j in range(_GKS):
            ix = idx_ref[step * _GKS + j]
            pltpu.make_async_copy(
                mc_ref.at[ix], buf.at[j], sem.at[sid]).start()

    def _compute(buf):
        pos = pos_ref[...]
        ctile = ((jax.lax.broadcasted_iota(jnp.int32, (_C, _R), 1) % _C)
                 == jax.lax.broadcasted_iota(jnp.int32, (_C, _R), 0)
                 ).astype(jnp.float32)
        for nh in range(_NH):
            chunks = buf[nh * _GK:(nh + 1) * _GK].reshape(_R, _D)
            u2 = u_ref[:, nh * _GH8:(nh + 1) * _GH8, :].reshape(_GH, _D)
            bqh = beta_ref[nh * _GH8:(nh + 1) * _GH8, :]
            w3 = w3_ref[nh * _GH8:(nh + 1) * _GH8, :]
            out_ref[nh * _GH8:(nh + 1) * _GH8, :] = _half_attend(
                chunks, pos, ctile, u2, bqh, w3, seg64_ref[...],
                seg64t_ref[...], vld_ref[...], wva_ref[...], bva_ref[...],
                boa_ref[...], woa_ref[...])

    @pl.when(i == 0)
    def _():
        _issue(i, abuf0, 0)

    @pl.when(jax.lax.rem(i, 2) == 0)
    def _():
        pltpu.make_async_copy(
            mc_ref.at[pl.ds(0, _GKS)], abuf0, sem.at[0]).wait()

        @pl.when(i + 1 < _NSTEP)
        def _():
            _issue(i + 1, abuf1, 1)

        _compute(abuf0)

    @pl.when(jax.lax.rem(i, 2) == 1)
    def _():
        pltpu.make_async_copy(
            mc_ref.at[pl.ds(0, _GKS)], abuf1, sem.at[1]).wait()

        @pl.when(i + 1 < _NSTEP)
        def _():
            _issue(i + 1, abuf0, 0)

        _compute(abuf1)


def _attend_call(flat_idx, mc_flat, pos, u, beta, w2, seg64, seg64t, vld,
                 Wv_a, bv2, Wo_a, bo2):
    grid_spec = pltpu.PrefetchScalarGridSpec(
        num_scalar_prefetch=1,
        grid=(_NSTEP,),
        in_specs=[
            pl.BlockSpec(memory_space=pl.ANY),
            pl.BlockSpec((_C, _D), lambda g, idx_ref: (0, 0)),
            pl.BlockSpec((_H, _G, _D), lambda g, idx_ref: (0, g, 0)),
            pl.BlockSpec((_G, _H), lambda g, idx_ref: (g, 0)),
            pl.BlockSpec((_G, _K), lambda g, idx_ref: (g, 0)),
            pl.BlockSpec((_GK, _R), lambda g, idx_ref: (0, 0)),
            pl.BlockSpec((_R, _GK), lambda g, idx_ref: (0, 0)),
            pl.BlockSpec((_GH, _R), lambda g, idx_ref: (0, 0)),
            pl.BlockSpec((_D, _S), lambda g, idx_ref: (0, 0)),
            pl.BlockSpec((1, _S), lambda g, idx_ref: (0, 0)),
            pl.BlockSpec((_S, _S), lambda g, idx_ref: (0, 0)),
            pl.BlockSpec((1, _S), lambda g, idx_ref: (0, 0)),
        ],
        out_specs=pl.BlockSpec((_G, _S), lambda g, idx_ref: (g, 0)),
        scratch_shapes=[
            pltpu.VMEM((_GKS, _C, _D), jnp.float32),
            pltpu.VMEM((_GKS, _C, _D), jnp.float32),
            pltpu.SemaphoreType.DMA((2,)),
        ],
    )
    return pl.pallas_call(
        _attend_body,
        grid_spec=grid_spec,
        out_shape=jax.ShapeDtypeStruct((_B * _Q, _S), jnp.float32),
    )(flat_idx, mc_flat, pos, u, beta, w2, seg64, seg64t, vld,
      Wv_a, bv2, Wo_a, bo2)


def _np_consts():
    r = np.arange(_R)
    seg64 = (r[None, :] // _C == np.arange(_GK)[:, None]).astype(np.float32)
    seg64t = seg64.T.copy()
    j = np.arange(_GH)
    vld = ((j[:, None] % _GH8)
           == (r[None, :] // (_K * _C))).astype(np.float32)
    return seg64, seg64t, vld


def kernel(queries, memory_keys, memory_contents, steps_since_last_write,
           accumulator, Wq, Wk, Wq_a, bq_a, Wk_a, bk_a, Wv_a, bv_a, Wo_a,
           bo_a):
    del steps_since_last_write, accumulator
    idx, w, u, beta = _select_call(
        queries, memory_keys, Wq, Wk, Wq_a, bq_a.reshape(1, _S), Wk_a,
        bk_a.reshape(1, _S))
    mc_flat = memory_contents.reshape(_B * _M, _C, _D)
    flat_idx = idx.reshape(-1)
    pos = jnp.asarray(_pos_enc_np())
    seg64, seg64t, vld = _np_consts()
    out = _attend_call(flat_idx, mc_flat, pos, u, beta,
                       w.reshape(_B * _Q, _K), jnp.asarray(seg64),
                       jnp.asarray(seg64t), jnp.asarray(vld), Wv_a,
                       bv_a.reshape(1, _S), Wo_a, bo_a.reshape(1, _S))
    return out.reshape(_B, _Q, _S)
